# baseline (speedup 1.0000x reference)
import functools
import math

import jax
import jax.numpy as jnp
from jax import lax
from jax.experimental import pallas as pl
from jax.experimental.pallas import tpu as pltpu
from jax.experimental.pallas import tpu_sc as plsc

N = 10000
E = 160000
NUM_BASIS = 32
MAX_RADIUS = 2.0
DIM = 288
HALF = 144
_SILU_CST = 1.6791767923989418

NPAD = 10240
_F32 = jnp.float32


def _atom_body(z_ref, wtA_ref, wtB_ref, bA_ref, bB_ref, oA_ref, oB_ref):
  zb = z_ref[...]
  oA_ref[...] = jnp.dot(zb, wtA_ref[...], preferred_element_type=_F32) + bA_ref[...]
  oB_ref[...] = jnp.dot(zb, wtB_ref[...], preferred_element_type=_F32) + bB_ref[...]


def _atom_tables(z, wtA, wtB, bA, bB):
  bl = 1000
  grid = N // bl
  return pl.pallas_call(
      _atom_body,
      grid=(grid,),
      in_specs=[
          pl.BlockSpec((bl, 4), lambda i: (i, 0)),
          pl.BlockSpec((4, HALF), lambda i: (0, 0)),
          pl.BlockSpec((4, HALF), lambda i: (0, 0)),
          pl.BlockSpec((1, HALF), lambda i: (0, 0)),
          pl.BlockSpec((1, HALF), lambda i: (0, 0)),
      ],
      out_specs=[
          pl.BlockSpec((bl, HALF), lambda i: (i, 0)),
          pl.BlockSpec((bl, HALF), lambda i: (i, 0)),
      ],
      out_shape=[
          jax.ShapeDtypeStruct((N, HALF), _F32),
          jax.ShapeDtypeStruct((N, HALF), _F32),
      ],
  )(z, wtA, wtB, bA, bB)


_EPT = E // 32


def _vec_body(pos_x, pos_y, pos_z, srcs, dsts, vx_o, vy_o, vz_o,
              px, py, pz, sv, dv, ox, oy, oz):
  cid = lax.axis_index("c")
  sid = lax.axis_index("s")
  base = (cid * 16 + sid) * _EPT
  pltpu.sync_copy(pos_x, px)
  pltpu.sync_copy(pos_y, py)
  pltpu.sync_copy(pos_z, pz)
  pltpu.sync_copy(srcs.at[pl.ds(base, _EPT)], sv)
  pltpu.sync_copy(dsts.at[pl.ds(base, _EPT)], dv)

  def group(off):
    si = sv[pl.ds(off, 16)]
    di = dv[pl.ds(off, 16)]
    for pref, ob in ((px, ox), (py, oy), (pz, oz)):
      a = plsc.load_gather(pref, [si])
      b = plsc.load_gather(pref, [di])
      ob[pl.ds(off, 16)] = a - b

  @pl.loop(0, _EPT // 16)
  def _(g):
    group(g * 16)

  group(_EPT - 16)

  pltpu.sync_copy(ox, vx_o.at[pl.ds(base, _EPT)])
  pltpu.sync_copy(oy, vy_o.at[pl.ds(base, _EPT)])
  pltpu.sync_copy(oz, vz_o.at[pl.ds(base, _EPT)])


def _edge_vec(pos_x, pos_y, pos_z, srcs, dsts):
  mesh = plsc.VectorSubcoreMesh(core_axis_name="c", subcore_axis_name="s")
  f = pl.kernel(
      _vec_body,
      out_type=(
          jax.ShapeDtypeStruct((E,), _F32),
          jax.ShapeDtypeStruct((E,), _F32),
          jax.ShapeDtypeStruct((E,), _F32),
      ),
      mesh=mesh,
      scratch_types=[
          pltpu.VMEM((N,), _F32),
          pltpu.VMEM((N,), _F32),
          pltpu.VMEM((N,), _F32),
          pltpu.VMEM((_EPT,), jnp.int32),
          pltpu.VMEM((_EPT,), jnp.int32),
          pltpu.VMEM((_EPT,), _F32),
          pltpu.VMEM((_EPT,), _F32),
          pltpu.VMEM((_EPT,), _F32),
      ],
      compiler_params=pltpu.CompilerParams(needs_layout_passes=False),
  )
  return f(pos_x, pos_y, pos_z, srcs, dsts)


_EBL = 1280


def _dense_body(vx_ref, vy_ref, vz_ref, St0T_ref, St1T_ref, St2T_ref,
                W1sT_ref, W2s0T_ref, W2s1T_ref, W2s2T_ref,
                o0_ref, o1_ref, o2_ref):
  x = vx_ref[0]
  y = vy_ref[0]
  zc = vz_ref[0]
  vl = jnp.sqrt(x * x + y * y + zc * zc + 1e-18)
  inv = 1.0 / vl
  ux = x * inv
  uy = y * inv
  uz = zc * inv
  s3 = math.sqrt(3.0)
  s5 = math.sqrt(5.0)
  x2 = ux * ux
  y2 = uy * uy
  z2 = uz * uz
  sh9T = jnp.concatenate([
      jnp.ones_like(vl),
      s3 * ux, s3 * uy, s3 * uz,
      s5 * (s3 * ux * uz),
      s5 * (s3 * ux * uy),
      s5 * (y2 - 0.5 * (x2 + z2)),
      s5 * (s3 * uy * uz),
      s5 * (0.5 * s3 * (z2 - x2)),
  ], axis=0)
  vlT = vl
  roots = ((lax.broadcasted_iota(jnp.int32, (NUM_BASIS, 1), 0).astype(_F32)
            + 1.0) * math.pi)
  safe_r = jnp.where(vlT > 1e-9, vlT, 1.0)
  mask = ((vlT < MAX_RADIUS) & (vlT > 0)).astype(_F32)
  coef = jnp.sqrt(2.0 / MAX_RADIUS) * mask / safe_r
  rbT = jnp.sin(roots * (vlT / MAX_RADIUS)) * coef
  hT = _SILU_CST * jax.nn.silu(
      jnp.dot(W1sT_ref[...], rbT, preferred_element_type=_F32))
  out0T = (jnp.dot(W2s0T_ref[...], hT, preferred_element_type=_F32) *
           jnp.dot(St0T_ref[...], sh9T, preferred_element_type=_F32))
  out1T = (jnp.dot(W2s1T_ref[...], hT, preferred_element_type=_F32) *
           jnp.dot(St1T_ref[...], sh9T, preferred_element_type=_F32))
  out2T = (jnp.dot(W2s2T_ref[...], hT, preferred_element_type=_F32) *
           jnp.dot(St2T_ref[...], sh9T, preferred_element_type=_F32))
  o0_ref[...] = out0T.T
  o1_ref[...] = out1T.T
  o2_ref[...] = jnp.concatenate(
      [out2T.T, jnp.zeros((_EBL, 96), _F32)], axis=1)


def _edge_dense(vx, vy, vz, St0T, St1T, St2T, W1sT, W2s0T, W2s1T, W2s2T):
  grid = E // _EBL
  vx = vx.reshape(grid, 1, _EBL)
  vy = vy.reshape(grid, 1, _EBL)
  vz = vz.reshape(grid, 1, _EBL)
  return pl.pallas_call(
      _dense_body,
      grid=(grid,),
      in_specs=[
          pl.BlockSpec((1, 1, _EBL), lambda i: (i, 0, 0)),
          pl.BlockSpec((1, 1, _EBL), lambda i: (i, 0, 0)),
          pl.BlockSpec((1, 1, _EBL), lambda i: (i, 0, 0)),
          pl.BlockSpec((128, 9), lambda i: (0, 0)),
          pl.BlockSpec((128, 9), lambda i: (0, 0)),
          pl.BlockSpec((32, 9), lambda i: (0, 0)),
          pl.BlockSpec((NUM_BASIS, NUM_BASIS), lambda i: (0, 0)),
          pl.BlockSpec((128, NUM_BASIS), lambda i: (0, 0)),
          pl.BlockSpec((128, NUM_BASIS), lambda i: (0, 0)),
          pl.BlockSpec((32, NUM_BASIS), lambda i: (0, 0)),
      ],
      out_specs=[
          pl.BlockSpec((_EBL, 128), lambda i: (i, 0)),
          pl.BlockSpec((_EBL, 128), lambda i: (i, 0)),
          pl.BlockSpec((_EBL, 128), lambda i: (i, 0)),
      ],
      out_shape=[
          jax.ShapeDtypeStruct((E, 128), _F32),
          jax.ShapeDtypeStruct((E, 128), _F32),
          jax.ShapeDtypeStruct((E, 128), _F32),
      ],
  )(vx, vy, vz, St0T, St1T, St2T, W1sT, W2s0T, W2s1T, W2s2T)


_K = 40
_KR = _K * 9 // 8
_NCH = (E // 16) // _K
_EPS = E // 16


def _scatter_body(atomA, atomB, R0, R1, R2, srcs, dsts, outA, outB,
                  acc, zb,
                  ss0, ds0, gb0, rx0, ry0, ss1, ds1, gb1, rx1, ry1,
                  sg0, sr0, st0, sg1, sr1, st1):
  cid = lax.axis_index("c")
  sid = lax.axis_index("s")

  @pl.loop(0, _K)
  def _(i):
    for k in range(HALF // 16):
      zb[i, pl.ds(k * 16, 16)] = jnp.zeros((16,), _F32)

  @pl.loop(0, NPAD // 16 // _K)
  def _(t):
    pltpu.sync_copy(zb, acc.at[pl.ds(sid * (NPAD // 16) + t * _K, _K)])
  plsc.subcore_barrier()

  base = sid * _EPS
  bufs = ((ss0, ds0, gb0, rx0, ry0, sg0, sr0, st0),
          (ss1, ds1, gb1, rx1, ry1, sg1, sr1, st1))

  def run(at_, ot_, rmain, rm_lo, rm_w, rtail, rt_lo, rt_w, kmap):
    def _mdesc(g, bk):
      ss, dd, gb, rx, ry, sg, sr, st = bk
      off = base + g * _K
      m = pltpu.make_async_copy(
          rmain.at[pl.ds(off, _K), pl.ds(rm_lo, rm_w)],
          rx.at[:, pl.ds(0, rm_w)], sr)
      t = pltpu.make_async_copy(
          rtail.at[pl.ds(off, _K), pl.ds(rt_lo, rt_w)],
          ry.at[:, pl.ds(0, rt_w)], st)
      return m, t

    def issue(g, bk):
      ss, dd, gb, rx, ry, sg, sr, st = bk
      off = base + g * _K
      pltpu.sync_copy(srcs.at[pl.ds(off, _K)], ss)
      pltpu.sync_copy(dsts.at[pl.ds(off, _K)], dd)
      pltpu.make_async_copy(at_.at[ss], gb, sg).start()
      m, t = _mdesc(g, bk)
      m.start()
      t.start()

    def process(g, bk):
      ss, dd, gb, rx, ry, sg, sr, st = bk
      pltpu.make_async_copy(at_.at[ss], gb, sg).wait()
      m, t = _mdesc(g, bk)
      m.wait()
      t.wait()

      @pl.loop(0, _K)
      def _(i):
        for k, isx, lo in kmap:
          buf = rx if isx else ry
          gsl = pl.ds(k * 16, 16)
          gb[i, gsl] = gb[i, gsl] * buf[i, pl.ds(lo, 16)]

      pltpu.sync_copy(gb, acc.at[dd], add=True)

    issue(0, bufs[0])

    @pl.loop(0, _NCH - 2, step=2)
    def _(t):
      for b in range(2):
        g = t + b
        issue(g + 1, bufs[1 - b])
        process(g, bufs[b])

    issue(_NCH - 1, bufs[(_NCH - 1) % 2])
    process(_NCH - 2, bufs[(_NCH - 2) % 2])
    process(_NCH - 1, bufs[(_NCH - 1) % 2])
    plsc.subcore_barrier()

    @pl.loop(0, NPAD // 16 // _K)
    def _(t):
      row = sid * (NPAD // 16) + t * _K
      pltpu.sync_copy(acc.at[pl.ds(row, _K)], zb)
      pltpu.sync_copy(zb, ot_.at[pl.ds(row, _K)])

  kmap0 = [(k, True, 16 * k) for k in range(8)] + [(8, False, 0)]
  kmap1 = [(k, True, 16 * k) for k in range(7)] + [(7, False, 0), (8, False, 16)]

  @pl.when(cid == 0)
  def _():
    run(atomA, outA, R0, 0, 128, R1, 0, 16, kmap0)

  @pl.when(cid == 1)
  def _():
    run(atomB, outB, R1, 16, 112, R2, 0, 32, kmap1)


def _scatter(atomA, atomB, R0, R1, R2, srcs, dsts):
  mesh = plsc.VectorSubcoreMesh(core_axis_name="c", subcore_axis_name="s")
  f = pl.kernel(
      _scatter_body,
      out_type=(
          jax.ShapeDtypeStruct((NPAD, HALF), _F32),
          jax.ShapeDtypeStruct((NPAD, HALF), _F32),
      ),
      mesh=mesh,
      scratch_types=[
          pltpu.VMEM_SHARED((NPAD, HALF), _F32),
          pltpu.VMEM((_K, HALF), _F32),
          pltpu.VMEM((_K,), jnp.int32),
          pltpu.VMEM((_K,), jnp.int32),
          pltpu.VMEM((_K, HALF), _F32),
          pltpu.VMEM((_K, 128), _F32),
          pltpu.VMEM((_K, 32), _F32),
          pltpu.VMEM((_K,), jnp.int32),
          pltpu.VMEM((_K,), jnp.int32),
          pltpu.VMEM((_K, HALF), _F32),
          pltpu.VMEM((_K, 128), _F32),
          pltpu.VMEM((_K, 32), _F32),
          pltpu.SemaphoreType.DMA,
          pltpu.SemaphoreType.DMA,
          pltpu.SemaphoreType.DMA,
          pltpu.SemaphoreType.DMA,
          pltpu.SemaphoreType.DMA,
          pltpu.SemaphoreType.DMA,
      ],
      compiler_params=pltpu.CompilerParams(use_tc_tiling_on_sc=False),
  )
  return f(atomA, atomB, R0, R1, R2, srcs, dsts)


def kernel(pos, z, edge_index, W0, W1, W2, W_atom, b_atom, Wfc1, Wfc2):
  srcs = edge_index[0]
  dsts = edge_index[1]
  pos_x, pos_y, pos_z = pos[:, 0], pos[:, 1], pos[:, 2]

  St = jnp.zeros((9, DIM), _F32)
  St = St.at[0, 0:32].set(W0)
  St = St.at[1:4, 32:128].set(jnp.kron(W1[None, :], jnp.eye(3, dtype=_F32)))
  St = St.at[4:9, 128:288].set(jnp.kron(W2[None, :], jnp.eye(5, dtype=_F32)))
  St0T, St1T, St2T = St[:, :128].T, St[:, 128:256].T, St[:, 256:].T
  W1sT = (Wfc1 / math.sqrt(float(NUM_BASIS))).T
  W2s = Wfc2 / math.sqrt(32.0)
  W2s0T, W2s1T, W2s2T = W2s[:, :128].T, W2s[:, 128:256].T, W2s[:, 256:].T
  wt = W_atom.T
  wtA, wtB = wt[:, :HALF], wt[:, HALF:]
  bA, bB = b_atom[None, :HALF], b_atom[None, HALF:]

  atomA, atomB = _atom_tables(z, wtA, wtB, bA, bB)
  vx, vy, vz = _edge_vec(pos_x, pos_y, pos_z, srcs, dsts)
  R0, R1, R2 = _edge_dense(vx, vy, vz, St0T, St1T, St2T, W1sT,
                           W2s0T, W2s1T, W2s2T)
  outA, outB = _scatter(atomA, atomB, R0, R1, R2, srcs, dsts)
  return jnp.concatenate([outA[:N], outB[:N]], axis=1)


# speedup vs baseline: 1.0409x; 1.0409x over previous
import functools
import math

import jax
import jax.numpy as jnp
from jax import lax
from jax.experimental import pallas as pl
from jax.experimental.pallas import tpu as pltpu
from jax.experimental.pallas import tpu_sc as plsc

N = 10000
E = 160000
NUM_BASIS = 32
MAX_RADIUS = 2.0
DIM = 288
HALF = 144
_SILU_CST = 1.6791767923989418

NPAD = 10240
_F32 = jnp.float32


def _atom_body(z_ref, wtA_ref, wtB_ref, bA_ref, bB_ref, oA_ref, oB_ref):
  zb = z_ref[...]
  oA_ref[...] = jnp.dot(zb, wtA_ref[...], preferred_element_type=_F32) + bA_ref[...]
  oB_ref[...] = jnp.dot(zb, wtB_ref[...], preferred_element_type=_F32) + bB_ref[...]


def _atom_tables(z, wtA, wtB, bA, bB):
  bl = 1000
  grid = N // bl
  return pl.pallas_call(
      _atom_body,
      grid=(grid,),
      in_specs=[
          pl.BlockSpec((bl, 4), lambda i: (i, 0)),
          pl.BlockSpec((4, HALF), lambda i: (0, 0)),
          pl.BlockSpec((4, HALF), lambda i: (0, 0)),
          pl.BlockSpec((1, HALF), lambda i: (0, 0)),
          pl.BlockSpec((1, HALF), lambda i: (0, 0)),
      ],
      out_specs=[
          pl.BlockSpec((bl, HALF), lambda i: (i, 0)),
          pl.BlockSpec((bl, HALF), lambda i: (i, 0)),
      ],
      out_shape=[
          jax.ShapeDtypeStruct((N, HALF), _F32),
          jax.ShapeDtypeStruct((N, HALF), _F32),
      ],
  )(z, wtA, wtB, bA, bB)


_EPT = E // 32


def _vec_body(pos_x, pos_y, pos_z, srcs, dsts, vx_o, vy_o, vz_o,
              px, py, pz, sv, dv, ox, oy, oz):
  cid = lax.axis_index("c")
  sid = lax.axis_index("s")
  base = (cid * 16 + sid) * _EPT
  pltpu.sync_copy(pos_x, px)
  pltpu.sync_copy(pos_y, py)
  pltpu.sync_copy(pos_z, pz)
  pltpu.sync_copy(srcs.at[pl.ds(base, _EPT)], sv)
  pltpu.sync_copy(dsts.at[pl.ds(base, _EPT)], dv)

  def group(off):
    si = sv[pl.ds(off, 16)]
    di = dv[pl.ds(off, 16)]
    for pref, ob in ((px, ox), (py, oy), (pz, oz)):
      a = plsc.load_gather(pref, [si])
      b = plsc.load_gather(pref, [di])
      ob[pl.ds(off, 16)] = a - b

  @pl.loop(0, _EPT // 16)
  def _(g):
    group(g * 16)

  group(_EPT - 16)

  pltpu.sync_copy(ox, vx_o.at[pl.ds(base, _EPT)])
  pltpu.sync_copy(oy, vy_o.at[pl.ds(base, _EPT)])
  pltpu.sync_copy(oz, vz_o.at[pl.ds(base, _EPT)])


def _edge_vec(pos_x, pos_y, pos_z, srcs, dsts):
  mesh = plsc.VectorSubcoreMesh(core_axis_name="c", subcore_axis_name="s")
  f = pl.kernel(
      _vec_body,
      out_type=(
          jax.ShapeDtypeStruct((E,), _F32),
          jax.ShapeDtypeStruct((E,), _F32),
          jax.ShapeDtypeStruct((E,), _F32),
      ),
      mesh=mesh,
      scratch_types=[
          pltpu.VMEM((N,), _F32),
          pltpu.VMEM((N,), _F32),
          pltpu.VMEM((N,), _F32),
          pltpu.VMEM((_EPT,), jnp.int32),
          pltpu.VMEM((_EPT,), jnp.int32),
          pltpu.VMEM((_EPT,), _F32),
          pltpu.VMEM((_EPT,), _F32),
          pltpu.VMEM((_EPT,), _F32),
      ],
      compiler_params=pltpu.CompilerParams(needs_layout_passes=False),
  )
  return f(pos_x, pos_y, pos_z, srcs, dsts)


_EBL = 1280


def _dense_body(vx_ref, vy_ref, vz_ref, St0T_ref, St1T_ref, St2T_ref,
                W1sT_ref, W2s0T_ref, W2s1T_ref, W2s2T_ref,
                o0_ref, o1_ref, o2_ref):
  x = vx_ref[0]
  y = vy_ref[0]
  zc = vz_ref[0]
  vl = jnp.sqrt(x * x + y * y + zc * zc + 1e-18)
  inv = 1.0 / vl
  ux = x * inv
  uy = y * inv
  uz = zc * inv
  s3 = math.sqrt(3.0)
  s5 = math.sqrt(5.0)
  x2 = ux * ux
  y2 = uy * uy
  z2 = uz * uz
  sh9T = jnp.concatenate([
      jnp.ones_like(vl),
      s3 * ux, s3 * uy, s3 * uz,
      s5 * (s3 * ux * uz),
      s5 * (s3 * ux * uy),
      s5 * (y2 - 0.5 * (x2 + z2)),
      s5 * (s3 * uy * uz),
      s5 * (0.5 * s3 * (z2 - x2)),
  ], axis=0)
  vlT = vl
  roots = ((lax.broadcasted_iota(jnp.int32, (NUM_BASIS, 1), 0).astype(_F32)
            + 1.0) * math.pi)
  safe_r = jnp.where(vlT > 1e-9, vlT, 1.0)
  mask = ((vlT < MAX_RADIUS) & (vlT > 0)).astype(_F32)
  coef = jnp.sqrt(2.0 / MAX_RADIUS) * mask / safe_r
  rbT = jnp.sin(roots * (vlT / MAX_RADIUS)) * coef
  hT = _SILU_CST * jax.nn.silu(
      jnp.dot(W1sT_ref[...], rbT, preferred_element_type=_F32))
  out0T = (jnp.dot(W2s0T_ref[...], hT, preferred_element_type=_F32) *
           jnp.dot(St0T_ref[...], sh9T, preferred_element_type=_F32))
  out1T = (jnp.dot(W2s1T_ref[...], hT, preferred_element_type=_F32) *
           jnp.dot(St1T_ref[...], sh9T, preferred_element_type=_F32))
  out2T = (jnp.dot(W2s2T_ref[...], hT, preferred_element_type=_F32) *
           jnp.dot(St2T_ref[...], sh9T, preferred_element_type=_F32))
  o0_ref[...] = out0T.T
  o1_ref[...] = out1T.T
  o2_ref[...] = jnp.concatenate(
      [out2T.T, jnp.zeros((_EBL, 96), _F32)], axis=1)


def _edge_dense(vx, vy, vz, St0T, St1T, St2T, W1sT, W2s0T, W2s1T, W2s2T):
  grid = E // _EBL
  vx = vx.reshape(grid, 1, _EBL)
  vy = vy.reshape(grid, 1, _EBL)
  vz = vz.reshape(grid, 1, _EBL)
  return pl.pallas_call(
      _dense_body,
      grid=(grid,),
      in_specs=[
          pl.BlockSpec((1, 1, _EBL), lambda i: (i, 0, 0)),
          pl.BlockSpec((1, 1, _EBL), lambda i: (i, 0, 0)),
          pl.BlockSpec((1, 1, _EBL), lambda i: (i, 0, 0)),
          pl.BlockSpec((128, 9), lambda i: (0, 0)),
          pl.BlockSpec((128, 9), lambda i: (0, 0)),
          pl.BlockSpec((32, 9), lambda i: (0, 0)),
          pl.BlockSpec((NUM_BASIS, NUM_BASIS), lambda i: (0, 0)),
          pl.BlockSpec((128, NUM_BASIS), lambda i: (0, 0)),
          pl.BlockSpec((128, NUM_BASIS), lambda i: (0, 0)),
          pl.BlockSpec((32, NUM_BASIS), lambda i: (0, 0)),
      ],
      out_specs=[
          pl.BlockSpec((_EBL, 128), lambda i: (i, 0)),
          pl.BlockSpec((_EBL, 128), lambda i: (i, 0)),
          pl.BlockSpec((_EBL, 128), lambda i: (i, 0)),
      ],
      out_shape=[
          jax.ShapeDtypeStruct((E, 128), _F32),
          jax.ShapeDtypeStruct((E, 128), _F32),
          jax.ShapeDtypeStruct((E, 128), _F32),
      ],
  )(vx, vy, vz, St0T, St1T, St2T, W1sT, W2s0T, W2s1T, W2s2T)


_K = 40
_KR = _K * 9 // 8
_NCH = (E // 16) // _K
_EPS = E // 16


def _scatter_body(atomA, atomB, R0, R1, R2, srcs, dsts, outA, outB,
                  acc, zb,
                  ss0, ds0, gb0, rx0, ry0, ss1, ds1, gb1, rx1, ry1,
                  sg0, sr0, st0, sg1, sr1, st1):
  cid = lax.axis_index("c")
  sid = lax.axis_index("s")

  @pl.loop(0, _K)
  def _(i):
    for k in range(HALF // 16):
      zb[i, pl.ds(k * 16, 16)] = jnp.zeros((16,), _F32)

  @pl.loop(0, NPAD // 16 // _K)
  def _(t):
    pltpu.sync_copy(zb, acc.at[pl.ds(sid * (NPAD // 16) + t * _K, _K)])
  plsc.subcore_barrier()

  base = sid * _EPS
  bufs = ((ss0, ds0, gb0, rx0, ry0, sg0, sr0, st0),
          (ss1, ds1, gb1, rx1, ry1, sg1, sr1, st1))

  def run(at_, ot_, rmain, rtail, kmap):
    def _mdesc(g, bk):
      ss, dd, gb, rx, ry, sg, sr, st = bk
      off = base + g * _K
      m = pltpu.make_async_copy(rmain.at[pl.ds(off, _K)], rx, sr)
      t = pltpu.make_async_copy(rtail.at[pl.ds(off, _K)], ry, st)
      return m, t

    def issue(g, bk):
      ss, dd, gb, rx, ry, sg, sr, st = bk
      off = base + g * _K
      pltpu.sync_copy(srcs.at[pl.ds(off, _K)], ss)
      pltpu.sync_copy(dsts.at[pl.ds(off, _K)], dd)
      pltpu.make_async_copy(at_.at[ss], gb, sg).start()
      m, t = _mdesc(g, bk)
      m.start()
      t.start()

    def process(g, bk):
      ss, dd, gb, rx, ry, sg, sr, st = bk
      pltpu.make_async_copy(at_.at[ss], gb, sg).wait()
      m, t = _mdesc(g, bk)
      m.wait()
      t.wait()

      @pl.loop(0, _K)
      def _(i):
        for k, isx, lo in kmap:
          buf = rx if isx else ry
          gsl = pl.ds(k * 16, 16)
          gb[i, gsl] = gb[i, gsl] * buf[i, pl.ds(lo, 16)]

      pltpu.sync_copy(gb, acc.at[dd], add=True)

    issue(0, bufs[0])

    @pl.loop(0, _NCH - 2, step=2)
    def _(t):
      for b in range(2):
        g = t + b
        issue(g + 1, bufs[1 - b])
        process(g, bufs[b])

    issue(_NCH - 1, bufs[(_NCH - 1) % 2])
    process(_NCH - 2, bufs[(_NCH - 2) % 2])
    process(_NCH - 1, bufs[(_NCH - 1) % 2])
    plsc.subcore_barrier()

    @pl.loop(0, NPAD // 16 // _K)
    def _(t):
      row = sid * (NPAD // 16) + t * _K
      pltpu.sync_copy(acc.at[pl.ds(row, _K)], zb)
      pltpu.sync_copy(zb, ot_.at[pl.ds(row, _K)])

  kmap0 = [(k, True, 16 * k) for k in range(8)] + [(8, False, 0)]
  kmap1 = ([(k, True, 16 + 16 * k) for k in range(7)] +
           [(7, False, 0), (8, False, 16)])

  @pl.when(cid == 0)
  def _():
    run(atomA, outA, R0, R1, kmap0)

  @pl.when(cid == 1)
  def _():
    run(atomB, outB, R1, R2, kmap1)


def _scatter(atomA, atomB, R0, R1, R2, srcs, dsts):
  mesh = plsc.VectorSubcoreMesh(core_axis_name="c", subcore_axis_name="s")
  f = pl.kernel(
      _scatter_body,
      out_type=(
          jax.ShapeDtypeStruct((NPAD, HALF), _F32),
          jax.ShapeDtypeStruct((NPAD, HALF), _F32),
      ),
      mesh=mesh,
      scratch_types=[
          pltpu.VMEM_SHARED((NPAD, HALF), _F32),
          pltpu.VMEM((_K, HALF), _F32),
          pltpu.VMEM((_K,), jnp.int32),
          pltpu.VMEM((_K,), jnp.int32),
          pltpu.VMEM((_K, HALF), _F32),
          pltpu.VMEM((_K, 128), _F32),
          pltpu.VMEM((_K, 128), _F32),
          pltpu.VMEM((_K,), jnp.int32),
          pltpu.VMEM((_K,), jnp.int32),
          pltpu.VMEM((_K, HALF), _F32),
          pltpu.VMEM((_K, 128), _F32),
          pltpu.VMEM((_K, 128), _F32),
          pltpu.SemaphoreType.DMA,
          pltpu.SemaphoreType.DMA,
          pltpu.SemaphoreType.DMA,
          pltpu.SemaphoreType.DMA,
          pltpu.SemaphoreType.DMA,
          pltpu.SemaphoreType.DMA,
      ],
      compiler_params=pltpu.CompilerParams(use_tc_tiling_on_sc=False),
  )
  return f(atomA, atomB, R0, R1, R2, srcs, dsts)


def kernel(pos, z, edge_index, W0, W1, W2, W_atom, b_atom, Wfc1, Wfc2):
  srcs = edge_index[0]
  dsts = edge_index[1]
  pos_x, pos_y, pos_z = pos[:, 0], pos[:, 1], pos[:, 2]

  St = jnp.zeros((9, DIM), _F32)
  St = St.at[0, 0:32].set(W0)
  St = St.at[1:4, 32:128].set(jnp.kron(W1[None, :], jnp.eye(3, dtype=_F32)))
  St = St.at[4:9, 128:288].set(jnp.kron(W2[None, :], jnp.eye(5, dtype=_F32)))
  St0T, St1T, St2T = St[:, :128].T, St[:, 128:256].T, St[:, 256:].T
  W1sT = (Wfc1 / math.sqrt(float(NUM_BASIS))).T
  W2s = Wfc2 / math.sqrt(32.0)
  W2s0T, W2s1T, W2s2T = W2s[:, :128].T, W2s[:, 128:256].T, W2s[:, 256:].T
  wt = W_atom.T
  wtA, wtB = wt[:, :HALF], wt[:, HALF:]
  bA, bB = b_atom[None, :HALF], b_atom[None, HALF:]

  atomA, atomB = _atom_tables(z, wtA, wtB, bA, bB)
  vx, vy, vz = _edge_vec(pos_x, pos_y, pos_z, srcs, dsts)
  R0, R1, R2 = _edge_dense(vx, vy, vz, St0T, St1T, St2T, W1sT,
                           W2s0T, W2s1T, W2s2T)
  outA, outB = _scatter(atomA, atomB, R0, R1, R2, srcs, dsts)
  return jnp.concatenate([outA[:N], outB[:N]], axis=1)


# speedup vs baseline: 1.1464x; 1.1013x over previous
import functools
import math

import jax
import jax.numpy as jnp
from jax import lax
from jax.experimental import pallas as pl
from jax.experimental.pallas import tpu as pltpu
from jax.experimental.pallas import tpu_sc as plsc

N = 10000
E = 160000
NUM_BASIS = 32
MAX_RADIUS = 2.0
DIM = 288
HALF = 144
_SILU_CST = 1.6791767923989418

NPAD = 10240
_F32 = jnp.float32


def _atom_body(z_ref, wtA_ref, wtB_ref, bA_ref, bB_ref, oA_ref, oB_ref):
  zb = z_ref[...]
  oA_ref[...] = jnp.dot(zb, wtA_ref[...], preferred_element_type=_F32) + bA_ref[...]
  oB_ref[...] = jnp.dot(zb, wtB_ref[...], preferred_element_type=_F32) + bB_ref[...]


def _atom_tables(z, wtA, wtB, bA, bB):
  bl = 1000
  grid = N // bl
  return pl.pallas_call(
      _atom_body,
      grid=(grid,),
      in_specs=[
          pl.BlockSpec((bl, 4), lambda i: (i, 0)),
          pl.BlockSpec((4, HALF), lambda i: (0, 0)),
          pl.BlockSpec((4, HALF), lambda i: (0, 0)),
          pl.BlockSpec((1, HALF), lambda i: (0, 0)),
          pl.BlockSpec((1, HALF), lambda i: (0, 0)),
      ],
      out_specs=[
          pl.BlockSpec((bl, HALF), lambda i: (i, 0)),
          pl.BlockSpec((bl, HALF), lambda i: (i, 0)),
      ],
      out_shape=[
          jax.ShapeDtypeStruct((N, HALF), _F32),
          jax.ShapeDtypeStruct((N, HALF), _F32),
      ],
  )(z, wtA, wtB, bA, bB)


_EPT = E // 32


def _vec_body(pos_x, pos_y, pos_z, srcs, dsts, vx_o, vy_o, vz_o,
              px, py, pz, sv, dv, ox, oy, oz):
  cid = lax.axis_index("c")
  sid = lax.axis_index("s")
  base = (cid * 16 + sid) * _EPT
  pltpu.sync_copy(pos_x, px)
  pltpu.sync_copy(pos_y, py)
  pltpu.sync_copy(pos_z, pz)
  pltpu.sync_copy(srcs.at[pl.ds(base, _EPT)], sv)
  pltpu.sync_copy(dsts.at[pl.ds(base, _EPT)], dv)

  def group(off):
    si = sv[pl.ds(off, 16)]
    di = dv[pl.ds(off, 16)]
    for pref, ob in ((px, ox), (py, oy), (pz, oz)):
      a = plsc.load_gather(pref, [si])
      b = plsc.load_gather(pref, [di])
      ob[pl.ds(off, 16)] = a - b

  @pl.loop(0, _EPT // 16)
  def _(g):
    group(g * 16)

  group(_EPT - 16)

  pltpu.sync_copy(ox, vx_o.at[pl.ds(base, _EPT)])
  pltpu.sync_copy(oy, vy_o.at[pl.ds(base, _EPT)])
  pltpu.sync_copy(oz, vz_o.at[pl.ds(base, _EPT)])


def _edge_vec(pos_x, pos_y, pos_z, srcs, dsts):
  mesh = plsc.VectorSubcoreMesh(core_axis_name="c", subcore_axis_name="s")
  f = pl.kernel(
      _vec_body,
      out_type=(
          jax.ShapeDtypeStruct((E,), _F32),
          jax.ShapeDtypeStruct((E,), _F32),
          jax.ShapeDtypeStruct((E,), _F32),
      ),
      mesh=mesh,
      scratch_types=[
          pltpu.VMEM((N,), _F32),
          pltpu.VMEM((N,), _F32),
          pltpu.VMEM((N,), _F32),
          pltpu.VMEM((_EPT,), jnp.int32),
          pltpu.VMEM((_EPT,), jnp.int32),
          pltpu.VMEM((_EPT,), _F32),
          pltpu.VMEM((_EPT,), _F32),
          pltpu.VMEM((_EPT,), _F32),
      ],
      compiler_params=pltpu.CompilerParams(needs_layout_passes=False),
  )
  return f(pos_x, pos_y, pos_z, srcs, dsts)


_EBL = 1280


def _dense_body(vx_ref, vy_ref, vz_ref, St0T_ref, St1T_ref, St2T_ref,
                W1sT_ref, W2s0T_ref, W2s1T_ref, W2s2T_ref,
                o0_ref, o1_ref, o2_ref):
  x = vx_ref[0]
  y = vy_ref[0]
  zc = vz_ref[0]
  vl = jnp.sqrt(x * x + y * y + zc * zc + 1e-18)
  inv = 1.0 / vl
  ux = x * inv
  uy = y * inv
  uz = zc * inv
  s3 = math.sqrt(3.0)
  s5 = math.sqrt(5.0)
  x2 = ux * ux
  y2 = uy * uy
  z2 = uz * uz
  sh9T = jnp.concatenate([
      jnp.ones_like(vl),
      s3 * ux, s3 * uy, s3 * uz,
      s5 * (s3 * ux * uz),
      s5 * (s3 * ux * uy),
      s5 * (y2 - 0.5 * (x2 + z2)),
      s5 * (s3 * uy * uz),
      s5 * (0.5 * s3 * (z2 - x2)),
  ], axis=0)
  vlT = vl
  roots = ((lax.broadcasted_iota(jnp.int32, (NUM_BASIS, 1), 0).astype(_F32)
            + 1.0) * math.pi)
  safe_r = jnp.where(vlT > 1e-9, vlT, 1.0)
  mask = ((vlT < MAX_RADIUS) & (vlT > 0)).astype(_F32)
  coef = jnp.sqrt(2.0 / MAX_RADIUS) * mask / safe_r
  rbT = jnp.sin(roots * (vlT / MAX_RADIUS)) * coef
  hT = _SILU_CST * jax.nn.silu(
      jnp.dot(W1sT_ref[...], rbT, preferred_element_type=_F32))
  out0T = (jnp.dot(W2s0T_ref[...], hT, preferred_element_type=_F32) *
           jnp.dot(St0T_ref[...], sh9T, preferred_element_type=_F32))
  out1T = (jnp.dot(W2s1T_ref[...], hT, preferred_element_type=_F32) *
           jnp.dot(St1T_ref[...], sh9T, preferred_element_type=_F32))
  out2T = (jnp.dot(W2s2T_ref[...], hT, preferred_element_type=_F32) *
           jnp.dot(St2T_ref[...], sh9T, preferred_element_type=_F32))
  o0_ref[...] = out0T.T
  o1_ref[...] = out1T.T
  o2_ref[...] = jnp.concatenate(
      [out2T.T, jnp.zeros((_EBL, 96), _F32)], axis=1)


def _edge_dense(vx, vy, vz, St0T, St1T, St2T, W1sT, W2s0T, W2s1T, W2s2T):
  grid = E // _EBL
  vx = vx.reshape(grid, 1, _EBL)
  vy = vy.reshape(grid, 1, _EBL)
  vz = vz.reshape(grid, 1, _EBL)
  return pl.pallas_call(
      _dense_body,
      grid=(grid,),
      in_specs=[
          pl.BlockSpec((1, 1, _EBL), lambda i: (i, 0, 0)),
          pl.BlockSpec((1, 1, _EBL), lambda i: (i, 0, 0)),
          pl.BlockSpec((1, 1, _EBL), lambda i: (i, 0, 0)),
          pl.BlockSpec((128, 9), lambda i: (0, 0)),
          pl.BlockSpec((128, 9), lambda i: (0, 0)),
          pl.BlockSpec((32, 9), lambda i: (0, 0)),
          pl.BlockSpec((NUM_BASIS, NUM_BASIS), lambda i: (0, 0)),
          pl.BlockSpec((128, NUM_BASIS), lambda i: (0, 0)),
          pl.BlockSpec((128, NUM_BASIS), lambda i: (0, 0)),
          pl.BlockSpec((32, NUM_BASIS), lambda i: (0, 0)),
      ],
      out_specs=[
          pl.BlockSpec((_EBL, 128), lambda i: (i, 0)),
          pl.BlockSpec((_EBL, 128), lambda i: (i, 0)),
          pl.BlockSpec((_EBL, 128), lambda i: (i, 0)),
      ],
      out_shape=[
          jax.ShapeDtypeStruct((E, 128), _F32),
          jax.ShapeDtypeStruct((E, 128), _F32),
          jax.ShapeDtypeStruct((E, 128), _F32),
      ],
  )(vx, vy, vz, St0T, St1T, St2T, W1sT, W2s0T, W2s1T, W2s2T)


_K = 40
_KR = _K * 9 // 8
_NCH = (E // 16) // _K
_EPS = E // 16


_BCH = 50
_BE = _BCH * _K
_NB = _EPS // _BE


def _scatter_body(atomA, atomB, R0, R1, R2, srcs, dsts, outA, outB,
                  acc, svb,
                  dd0, gb0, rx0, ry0, dd1, gb1, rx1, ry1,
                  sg0, sr0, st0, sg1, sr1, st1):
  cid = lax.axis_index("c")
  sid = lax.axis_index("s")

  @pl.loop(0, _K)
  def _(i):
    for k in range(HALF // 16):
      gb0[i, pl.ds(k * 16, 16)] = jnp.zeros((16,), _F32)

  @pl.loop(0, NPAD // 16 // _K)
  def _(t):
    pltpu.sync_copy(gb0, acc.at[pl.ds(sid * (NPAD // 16) + t * _K, _K)])
  plsc.subcore_barrier()

  base = sid * _EPS
  bufs = ((dd0, gb0, rx0, ry0, sg0, sr0, st0),
          (dd1, gb1, rx1, ry1, sg1, sr1, st1))

  def run(at_, ot_, rmain, rtail, kmap):
    def gdesc(c, bk):
      dd, gb, rx, ry, sg, sr, st = bk
      return pltpu.make_async_copy(at_.at[svb.at[pl.ds(c * _K, _K)]], gb, sg)

    def issue(boff, c, bk):
      dd, gb, rx, ry, sg, sr, st = bk
      off = boff + c * _K
      pltpu.sync_copy(dsts.at[pl.ds(off, _K)], dd)
      gdesc(c, bk).start()
      pltpu.make_async_copy(rmain.at[pl.ds(off, _K)], rx, sr).start()
      pltpu.make_async_copy(rtail.at[pl.ds(off, _K)], ry, st).start()

    def process(boff, c, bk):
      dd, gb, rx, ry, sg, sr, st = bk
      off = boff + c * _K
      gdesc(c, bk).wait()
      pltpu.make_async_copy(rmain.at[pl.ds(off, _K)], rx, sr).wait()
      pltpu.make_async_copy(rtail.at[pl.ds(off, _K)], ry, st).wait()

      @pl.loop(0, _K)
      def _(i):
        for k, isx, lo in kmap:
          buf = rx if isx else ry
          gsl = pl.ds(k * 16, 16)
          gb[i, gsl] = gb[i, gsl] * buf[i, pl.ds(lo, 16)]

      pltpu.sync_copy(gb, acc.at[dd], add=True)

    @pl.loop(0, _NB)
    def _(b):
      boff = base + b * _BE
      pltpu.sync_copy(srcs.at[pl.ds(boff, _BE)], svb)
      issue(boff, 0, bufs[0])

      @pl.loop(0, _BCH - 2, step=2)
      def _(t):
        for b2 in range(2):
          g = t + b2
          issue(boff, g + 1, bufs[1 - b2])
          process(boff, g, bufs[b2])

      issue(boff, _BCH - 1, bufs[1])
      process(boff, _BCH - 2, bufs[0])
      process(boff, _BCH - 1, bufs[1])

    plsc.subcore_barrier()

    @pl.loop(0, NPAD // 16 // _K)
    def _(t):
      row = sid * (NPAD // 16) + t * _K
      pltpu.sync_copy(acc.at[pl.ds(row, _K)], gb0)
      pltpu.sync_copy(gb0, ot_.at[pl.ds(row, _K)])

  kmap0 = [(k, True, 16 * k) for k in range(8)] + [(8, False, 0)]
  kmap1 = ([(k, True, 16 + 16 * k) for k in range(7)] +
           [(7, False, 0), (8, False, 16)])

  @pl.when(cid == 0)
  def _():
    run(atomA, outA, R0, R1, kmap0)

  @pl.when(cid == 1)
  def _():
    run(atomB, outB, R1, R2, kmap1)


def _scatter(atomA, atomB, R0, R1, R2, srcs, dsts):
  mesh = plsc.VectorSubcoreMesh(core_axis_name="c", subcore_axis_name="s")
  f = pl.kernel(
      _scatter_body,
      out_type=(
          jax.ShapeDtypeStruct((NPAD, HALF), _F32),
          jax.ShapeDtypeStruct((NPAD, HALF), _F32),
      ),
      mesh=mesh,
      scratch_types=[
          pltpu.VMEM_SHARED((NPAD, HALF), _F32),
          pltpu.VMEM((_BE,), jnp.int32),
          pltpu.VMEM((_K,), jnp.int32),
          pltpu.VMEM((_K, HALF), _F32),
          pltpu.VMEM((_K, 128), _F32),
          pltpu.VMEM((_K, 128), _F32),
          pltpu.VMEM((_K,), jnp.int32),
          pltpu.VMEM((_K, HALF), _F32),
          pltpu.VMEM((_K, 128), _F32),
          pltpu.VMEM((_K, 128), _F32),
          pltpu.SemaphoreType.DMA,
          pltpu.SemaphoreType.DMA,
          pltpu.SemaphoreType.DMA,
          pltpu.SemaphoreType.DMA,
          pltpu.SemaphoreType.DMA,
          pltpu.SemaphoreType.DMA,
      ],
      compiler_params=pltpu.CompilerParams(use_tc_tiling_on_sc=False),
  )
  return f(atomA, atomB, R0, R1, R2, srcs, dsts)


def kernel(pos, z, edge_index, W0, W1, W2, W_atom, b_atom, Wfc1, Wfc2):
  srcs = edge_index[0]
  dsts = edge_index[1]
  pos_x, pos_y, pos_z = pos[:, 0], pos[:, 1], pos[:, 2]

  St = jnp.zeros((9, DIM), _F32)
  St = St.at[0, 0:32].set(W0)
  St = St.at[1:4, 32:128].set(jnp.kron(W1[None, :], jnp.eye(3, dtype=_F32)))
  St = St.at[4:9, 128:288].set(jnp.kron(W2[None, :], jnp.eye(5, dtype=_F32)))
  St0T, St1T, St2T = St[:, :128].T, St[:, 128:256].T, St[:, 256:].T
  W1sT = (Wfc1 / math.sqrt(float(NUM_BASIS))).T
  W2s = Wfc2 / math.sqrt(32.0)
  W2s0T, W2s1T, W2s2T = W2s[:, :128].T, W2s[:, 128:256].T, W2s[:, 256:].T
  wt = W_atom.T
  wtA, wtB = wt[:, :HALF], wt[:, HALF:]
  bA, bB = b_atom[None, :HALF], b_atom[None, HALF:]

  atomA, atomB = _atom_tables(z, wtA, wtB, bA, bB)
  vx, vy, vz = _edge_vec(pos_x, pos_y, pos_z, srcs, dsts)
  R0, R1, R2 = _edge_dense(vx, vy, vz, St0T, St1T, St2T, W1sT,
                           W2s0T, W2s1T, W2s2T)
  outA, outB = _scatter(atomA, atomB, R0, R1, R2, srcs, dsts)
  return jnp.concatenate([outA[:N], outB[:N]], axis=1)


# speedup vs baseline: 1.1545x; 1.0071x over previous
import functools
import math

import jax
import jax.numpy as jnp
from jax import lax
from jax.experimental import pallas as pl
from jax.experimental.pallas import tpu as pltpu
from jax.experimental.pallas import tpu_sc as plsc

N = 10000
E = 160000
NUM_BASIS = 32
MAX_RADIUS = 2.0
DIM = 288
HALF = 144
_SILU_CST = 1.6791767923989418

NPAD = 10240
_F32 = jnp.float32


def _atom_body(z_ref, wtA_ref, wtB_ref, bA_ref, bB_ref, oA_ref, oB_ref):
  zb = z_ref[...]
  oA_ref[...] = jnp.dot(zb, wtA_ref[...], preferred_element_type=_F32) + bA_ref[...]
  oB_ref[...] = jnp.dot(zb, wtB_ref[...], preferred_element_type=_F32) + bB_ref[...]


def _atom_tables(z, wtA, wtB, bA, bB):
  bl = 1000
  grid = N // bl
  return pl.pallas_call(
      _atom_body,
      grid=(grid,),
      in_specs=[
          pl.BlockSpec((bl, 4), lambda i: (i, 0)),
          pl.BlockSpec((4, HALF), lambda i: (0, 0)),
          pl.BlockSpec((4, HALF), lambda i: (0, 0)),
          pl.BlockSpec((1, HALF), lambda i: (0, 0)),
          pl.BlockSpec((1, HALF), lambda i: (0, 0)),
      ],
      out_specs=[
          pl.BlockSpec((bl, HALF), lambda i: (i, 0)),
          pl.BlockSpec((bl, HALF), lambda i: (i, 0)),
      ],
      out_shape=[
          jax.ShapeDtypeStruct((N, HALF), _F32),
          jax.ShapeDtypeStruct((N, HALF), _F32),
      ],
  )(z, wtA, wtB, bA, bB)


_EPT = E // 32


def _vec_body(pos_x, pos_y, pos_z, srcs, dsts, vx_o, vy_o, vz_o,
              px, py, pz, sv, dv, ox, oy, oz):
  cid = lax.axis_index("c")
  sid = lax.axis_index("s")
  base = (cid * 16 + sid) * _EPT
  pltpu.sync_copy(pos_x, px)
  pltpu.sync_copy(pos_y, py)
  pltpu.sync_copy(pos_z, pz)
  pltpu.sync_copy(srcs.at[pl.ds(base, _EPT)], sv)
  pltpu.sync_copy(dsts.at[pl.ds(base, _EPT)], dv)

  def group(off):
    si = sv[pl.ds(off, 16)]
    di = dv[pl.ds(off, 16)]
    for pref, ob in ((px, ox), (py, oy), (pz, oz)):
      a = plsc.load_gather(pref, [si])
      b = plsc.load_gather(pref, [di])
      ob[pl.ds(off, 16)] = a - b

  @pl.loop(0, _EPT // 16)
  def _(g):
    group(g * 16)

  group(_EPT - 16)

  pltpu.sync_copy(ox, vx_o.at[pl.ds(base, _EPT)])
  pltpu.sync_copy(oy, vy_o.at[pl.ds(base, _EPT)])
  pltpu.sync_copy(oz, vz_o.at[pl.ds(base, _EPT)])


def _edge_vec(pos_x, pos_y, pos_z, srcs, dsts):
  mesh = plsc.VectorSubcoreMesh(core_axis_name="c", subcore_axis_name="s")
  f = pl.kernel(
      _vec_body,
      out_type=(
          jax.ShapeDtypeStruct((E,), _F32),
          jax.ShapeDtypeStruct((E,), _F32),
          jax.ShapeDtypeStruct((E,), _F32),
      ),
      mesh=mesh,
      scratch_types=[
          pltpu.VMEM((N,), _F32),
          pltpu.VMEM((N,), _F32),
          pltpu.VMEM((N,), _F32),
          pltpu.VMEM((_EPT,), jnp.int32),
          pltpu.VMEM((_EPT,), jnp.int32),
          pltpu.VMEM((_EPT,), _F32),
          pltpu.VMEM((_EPT,), _F32),
          pltpu.VMEM((_EPT,), _F32),
      ],
      compiler_params=pltpu.CompilerParams(needs_layout_passes=False),
  )
  return f(pos_x, pos_y, pos_z, srcs, dsts)


_EBL = 1280


def _dense_body(vx_ref, vy_ref, vz_ref, St0T_ref, St1T_ref, St2T_ref,
                W1sT_ref, W2s0T_ref, W2s1T_ref, W2s2T_ref,
                o0_ref, o1_ref, o2_ref):
  x = vx_ref[0]
  y = vy_ref[0]
  zc = vz_ref[0]
  vl = jnp.sqrt(x * x + y * y + zc * zc + 1e-18)
  inv = 1.0 / vl
  ux = x * inv
  uy = y * inv
  uz = zc * inv
  s3 = math.sqrt(3.0)
  s5 = math.sqrt(5.0)
  x2 = ux * ux
  y2 = uy * uy
  z2 = uz * uz
  sh9T = jnp.concatenate([
      jnp.ones_like(vl),
      s3 * ux, s3 * uy, s3 * uz,
      s5 * (s3 * ux * uz),
      s5 * (s3 * ux * uy),
      s5 * (y2 - 0.5 * (x2 + z2)),
      s5 * (s3 * uy * uz),
      s5 * (0.5 * s3 * (z2 - x2)),
  ], axis=0)
  vlT = vl
  roots = ((lax.broadcasted_iota(jnp.int32, (NUM_BASIS, 1), 0).astype(_F32)
            + 1.0) * math.pi)
  safe_r = jnp.where(vlT > 1e-9, vlT, 1.0)
  mask = ((vlT < MAX_RADIUS) & (vlT > 0)).astype(_F32)
  coef = jnp.sqrt(2.0 / MAX_RADIUS) * mask / safe_r
  rbT = jnp.sin(roots * (vlT / MAX_RADIUS)) * coef
  hT = _SILU_CST * jax.nn.silu(
      jnp.dot(W1sT_ref[...], rbT, preferred_element_type=_F32))
  out0T = (jnp.dot(W2s0T_ref[...], hT, preferred_element_type=_F32) *
           jnp.dot(St0T_ref[...], sh9T, preferred_element_type=_F32))
  out1T = (jnp.dot(W2s1T_ref[...], hT, preferred_element_type=_F32) *
           jnp.dot(St1T_ref[...], sh9T, preferred_element_type=_F32))
  out2T = (jnp.dot(W2s2T_ref[...], hT, preferred_element_type=_F32) *
           jnp.dot(St2T_ref[...], sh9T, preferred_element_type=_F32))
  o0_ref[...] = out0T.T
  o1_ref[...] = out1T.T
  o2_ref[...] = jnp.concatenate(
      [out2T.T, jnp.zeros((_EBL, 96), _F32)], axis=1)


def _edge_dense(vx, vy, vz, St0T, St1T, St2T, W1sT, W2s0T, W2s1T, W2s2T):
  grid = E // _EBL
  vx = vx.reshape(grid, 1, _EBL)
  vy = vy.reshape(grid, 1, _EBL)
  vz = vz.reshape(grid, 1, _EBL)
  return pl.pallas_call(
      _dense_body,
      grid=(grid,),
      in_specs=[
          pl.BlockSpec((1, 1, _EBL), lambda i: (i, 0, 0)),
          pl.BlockSpec((1, 1, _EBL), lambda i: (i, 0, 0)),
          pl.BlockSpec((1, 1, _EBL), lambda i: (i, 0, 0)),
          pl.BlockSpec((128, 9), lambda i: (0, 0)),
          pl.BlockSpec((128, 9), lambda i: (0, 0)),
          pl.BlockSpec((32, 9), lambda i: (0, 0)),
          pl.BlockSpec((NUM_BASIS, NUM_BASIS), lambda i: (0, 0)),
          pl.BlockSpec((128, NUM_BASIS), lambda i: (0, 0)),
          pl.BlockSpec((128, NUM_BASIS), lambda i: (0, 0)),
          pl.BlockSpec((32, NUM_BASIS), lambda i: (0, 0)),
      ],
      out_specs=[
          pl.BlockSpec((_EBL, 128), lambda i: (i, 0)),
          pl.BlockSpec((_EBL, 128), lambda i: (i, 0)),
          pl.BlockSpec((_EBL, 128), lambda i: (i, 0)),
      ],
      out_shape=[
          jax.ShapeDtypeStruct((E, 128), _F32),
          jax.ShapeDtypeStruct((E, 128), _F32),
          jax.ShapeDtypeStruct((E, 128), _F32),
      ],
  )(vx, vy, vz, St0T, St1T, St2T, W1sT, W2s0T, W2s1T, W2s2T)


_K = 40
_KR = _K * 9 // 8
_NCH = (E // 16) // _K
_EPS = E // 16


_BCH = 50
_BE = _BCH * _K
_NB = _EPS // _BE


def _scatter_body(atomA, atomB, R0, R1, R2, srcs, dsts, out,
                  acc, svb,
                  dd0, gb0, rx0, ry0, dd1, gb1, rx1, ry1,
                  sg0, sr0, st0, sg1, sr1, st1):
  cid = lax.axis_index("c")
  sid = lax.axis_index("s")

  @pl.loop(0, _K)
  def _(i):
    for k in range(HALF // 16):
      gb0[i, pl.ds(k * 16, 16)] = jnp.zeros((16,), _F32)

  @pl.loop(0, NPAD // 16 // _K)
  def _(t):
    pltpu.sync_copy(gb0, acc.at[pl.ds(sid * (NPAD // 16) + t * _K, _K)])
  plsc.subcore_barrier()

  base = sid * _EPS
  bufs = ((dd0, gb0, rx0, ry0, sg0, sr0, st0),
          (dd1, gb1, rx1, ry1, sg1, sr1, st1))

  def run(at_, colo, rmain, rtail, kmap):
    def gdesc(c, bk):
      dd, gb, rx, ry, sg, sr, st = bk
      return pltpu.make_async_copy(at_.at[svb.at[pl.ds(c * _K, _K)]], gb, sg)

    def issue(boff, c, bk):
      dd, gb, rx, ry, sg, sr, st = bk
      off = boff + c * _K
      pltpu.sync_copy(dsts.at[pl.ds(off, _K)], dd)
      gdesc(c, bk).start()
      pltpu.make_async_copy(rmain.at[pl.ds(off, _K)], rx, sr).start()
      pltpu.make_async_copy(rtail.at[pl.ds(off, _K)], ry, st).start()

    def process(boff, c, bk):
      dd, gb, rx, ry, sg, sr, st = bk
      off = boff + c * _K
      gdesc(c, bk).wait()
      pltpu.make_async_copy(rmain.at[pl.ds(off, _K)], rx, sr).wait()
      pltpu.make_async_copy(rtail.at[pl.ds(off, _K)], ry, st).wait()

      @pl.loop(0, _K)
      def _(i):
        for k, isx, lo in kmap:
          buf = rx if isx else ry
          gsl = pl.ds(k * 16, 16)
          gb[i, gsl] = gb[i, gsl] * buf[i, pl.ds(lo, 16)]

      pltpu.sync_copy(gb, acc.at[dd], add=True)

    @pl.loop(0, _NB)
    def _(b):
      boff = base + b * _BE
      pltpu.sync_copy(srcs.at[pl.ds(boff, _BE)], svb)
      issue(boff, 0, bufs[0])

      @pl.loop(0, _BCH - 2, step=2)
      def _(t):
        for b2 in range(2):
          g = t + b2
          issue(boff, g + 1, bufs[1 - b2])
          process(boff, g, bufs[b2])

      issue(boff, _BCH - 1, bufs[1])
      process(boff, _BCH - 2, bufs[0])
      process(boff, _BCH - 1, bufs[1])

    plsc.subcore_barrier()

    @pl.loop(0, NPAD // 16 // _K)
    def _(t):
      row = sid * (NPAD // 16) + t * _K
      pltpu.sync_copy(acc.at[pl.ds(row, _K)], gb0)
      pltpu.sync_copy(gb0, out.at[pl.ds(row, _K), pl.ds(colo, HALF)])

  kmap0 = [(k, True, 16 * k) for k in range(8)] + [(8, False, 0)]
  kmap1 = ([(k, True, 16 + 16 * k) for k in range(7)] +
           [(7, False, 0), (8, False, 16)])

  @pl.when(cid == 0)
  def _():
    run(atomA, 0, R0, R1, kmap0)

  @pl.when(cid == 1)
  def _():
    run(atomB, HALF, R1, R2, kmap1)


def _scatter(atomA, atomB, R0, R1, R2, srcs, dsts):
  mesh = plsc.VectorSubcoreMesh(core_axis_name="c", subcore_axis_name="s")
  f = pl.kernel(
      _scatter_body,
      out_type=jax.ShapeDtypeStruct((NPAD, DIM), _F32),
      mesh=mesh,
      scratch_types=[
          pltpu.VMEM_SHARED((NPAD, HALF), _F32),
          pltpu.VMEM((_BE,), jnp.int32),
          pltpu.VMEM((_K,), jnp.int32),
          pltpu.VMEM((_K, HALF), _F32),
          pltpu.VMEM((_K, 128), _F32),
          pltpu.VMEM((_K, 128), _F32),
          pltpu.VMEM((_K,), jnp.int32),
          pltpu.VMEM((_K, HALF), _F32),
          pltpu.VMEM((_K, 128), _F32),
          pltpu.VMEM((_K, 128), _F32),
          pltpu.SemaphoreType.DMA,
          pltpu.SemaphoreType.DMA,
          pltpu.SemaphoreType.DMA,
          pltpu.SemaphoreType.DMA,
          pltpu.SemaphoreType.DMA,
          pltpu.SemaphoreType.DMA,
      ],
      compiler_params=pltpu.CompilerParams(use_tc_tiling_on_sc=False),
  )
  return f(atomA, atomB, R0, R1, R2, srcs, dsts)


def kernel(pos, z, edge_index, W0, W1, W2, W_atom, b_atom, Wfc1, Wfc2):
  srcs = edge_index[0]
  dsts = edge_index[1]
  pos_x, pos_y, pos_z = pos[:, 0], pos[:, 1], pos[:, 2]

  St = jnp.zeros((9, DIM), _F32)
  St = St.at[0, 0:32].set(W0)
  St = St.at[1:4, 32:128].set(jnp.kron(W1[None, :], jnp.eye(3, dtype=_F32)))
  St = St.at[4:9, 128:288].set(jnp.kron(W2[None, :], jnp.eye(5, dtype=_F32)))
  St0T, St1T, St2T = St[:, :128].T, St[:, 128:256].T, St[:, 256:].T
  W1sT = (Wfc1 / math.sqrt(float(NUM_BASIS))).T
  W2s = Wfc2 / math.sqrt(32.0)
  W2s0T, W2s1T, W2s2T = W2s[:, :128].T, W2s[:, 128:256].T, W2s[:, 256:].T
  wt = W_atom.T
  wtA, wtB = wt[:, :HALF], wt[:, HALF:]
  bA, bB = b_atom[None, :HALF], b_atom[None, HALF:]

  atomA, atomB = _atom_tables(z, wtA, wtB, bA, bB)
  vx, vy, vz = _edge_vec(pos_x, pos_y, pos_z, srcs, dsts)
  R0, R1, R2 = _edge_dense(vx, vy, vz, St0T, St1T, St2T, W1sT,
                           W2s0T, W2s1T, W2s2T)
  out = _scatter(atomA, atomB, R0, R1, R2, srcs, dsts)
  return out[:N]
